# baseline (speedup 1.0000x reference)
import functools
import math

import jax
import jax.numpy as jnp
from jax import lax
from jax.experimental import pallas as pl
from jax.experimental.pallas import tpu as pltpu
from jax.experimental.pallas import tpu_sc as plsc

D_MODEL = 64
SCALE = math.sqrt(D_MODEL)
NC, NS, L = 2, 16, 16
NW = NC * NS
RING = 4
OB = 2


def _sc_embed(x, table):
    n_seq, n_tok = x.shape
    rows_per_w = n_seq // NW
    n_iblk = n_seq // rows_per_w
    n_g8 = rows_per_w // L

    @functools.partial(
        pl.kernel,
        out_type=jax.ShapeDtypeStruct(
            (n_tok, D_MODEL // 8, n_iblk, 8, rows_per_w), jnp.float32
        ),
        mesh=plsc.VectorSubcoreMesh(
            core_axis_name="c", subcore_axis_name="s", num_cores=NC, num_subcores=NS
        ),
        scratch_types=[
            pltpu.VMEM((rows_per_w, n_tok), jnp.int32),
            pltpu.VMEM((RING, rows_per_w), jnp.int32),
            pltpu.VMEM((RING, rows_per_w, D_MODEL), jnp.float32),
            pltpu.VMEM((OB, D_MODEL // 8, 1, 8, rows_per_w), jnp.float32),
            pltpu.SemaphoreType.DMA((RING,)),
            pltpu.SemaphoreType.DMA((OB,)),
        ],
        compiler_params=pltpu.CompilerParams(
            use_tc_tiling_on_sc=False, needs_layout_passes=False
        ),
    )
    def k(x_hbm, tab_hbm, out_hbm, idx_v, cidx, gbuf, obuf, gsem, wsem):
        wid = lax.axis_index("s") * NC + lax.axis_index("c")
        i0 = wid * rows_per_w
        lanes = lax.iota(jnp.int32, L)
        rows8 = [lanes + (g8 * L) for g8 in range(n_g8)]

        pltpu.sync_copy(x_hbm.at[pl.ds(i0, rows_per_w)], idx_v)

        def fire(j, b):
            cols = jnp.zeros((L,), jnp.int32) + j
            for g8 in range(n_g8):
                cidx[b, pl.ds(g8 * L, L)] = plsc.load_gather(idx_v, [rows8[g8], cols])
            pltpu.async_copy(tab_hbm.at[cidx.at[b]], gbuf.at[b], gsem.at[b])

        def wait_gather(b):
            pltpu.make_async_copy(
                tab_hbm.at[cidx.at[b]], gbuf.at[b], gsem.at[b]
            ).wait()

        def out_slice(j):
            return out_hbm.at[j, pl.ds(0, D_MODEL // 8), pl.ds(wid, 1)]

        def start_write(j, ob):
            pltpu.async_copy(obuf.at[ob], out_slice(j), wsem.at[ob])

        def wait_write(j, ob):
            pltpu.make_async_copy(obuf.at[ob], out_slice(j), wsem.at[ob]).wait()

        for b in range(RING - 1):
            fire(b, b)

        @pl.loop(0, n_tok // RING)
        def _grp(g):
            for kk in range(RING):
                j = g * RING + kk
                ob = kk % OB
                wait_gather(kk)

                @pl.when(j >= OB)
                def _():
                    wait_write(j - OB, ob)

                gb = gbuf.at[kk]

                @plsc.parallel_loop(0, D_MODEL // 8, unroll=4)
                def _dt(dt):
                    for dr in range(8):
                        d = dt * 8 + dr
                        cols = jnp.zeros((L,), jnp.int32) + d
                        for g8 in range(n_g8):
                            vals = plsc.load_gather(gb, [rows8[g8], cols])
                            obuf[ob, dt, 0, dr, pl.ds(g8 * L, L)] = vals * SCALE

                start_write(j, ob)
                jn = j + RING - 1

                @pl.when(jn < n_tok)
                def _():
                    fire(jn, (kk + RING - 1) % RING)

        for ob in range(OB):
            wait_write(n_tok - OB + ob, ob)

    return k(x, table)


def kernel(x, table):
    out5 = _sc_embed(x.astype(jnp.int32), table)
    o = jnp.transpose(out5, (0, 1, 3, 2, 4))
    o = o.reshape(out5.shape[0], D_MODEL, -1)
    return jnp.transpose(o, (2, 0, 1))


# speedup vs baseline: 1.1184x; 1.1184x over previous
import functools
import math

import jax
import jax.numpy as jnp
from jax import lax
from jax.experimental import pallas as pl
from jax.experimental.pallas import tpu as pltpu
from jax.experimental.pallas import tpu_sc as plsc

D_MODEL = 64
SCALE = math.sqrt(D_MODEL)
NC, NS, L = 2, 16, 16
NW = NC * NS
RING = 4
HALF_OFF = (0, 128)
HALF_SZ = (128, 72)


def _sc_embed(x, table):
    n_seq, n_tok = x.shape
    rows_per_w = n_seq // NW
    nch = 2 * rows_per_w
    mesh = plsc.VectorSubcoreMesh(
        core_axis_name="c", subcore_axis_name="s", num_cores=NC, num_subcores=NS
    )

    @functools.partial(
        pl.kernel,
        out_type=jax.ShapeDtypeStruct((n_seq, n_tok, D_MODEL), jnp.float32),
        mesh=mesh,
        scratch_types=[
            pltpu.VMEM((rows_per_w, n_tok), jnp.int32),
            pltpu.VMEM((RING, HALF_SZ[0], D_MODEL), jnp.float32),
            pltpu.SemaphoreType.DMA((RING,)),
            pltpu.SemaphoreType.DMA((RING,)),
        ],
        compiler_params=pltpu.CompilerParams(use_tc_tiling_on_sc=False),
    )
    def k(x_hbm, tab_hbm, out_hbm, idx_v, rows_v, gsem, wsem):
        wid = lax.axis_index("s") * NC + lax.axis_index("c")
        xr0 = wid * rows_per_w

        pltpu.sync_copy(x_hbm.at[pl.ds(xr0, rows_per_w)], idx_v)

        def gather_pair(row, half, b):
            sz = HALF_SZ[half]
            idx = idx_v.at[row, pl.ds(HALF_OFF[half], sz)]
            return tab_hbm.at[idx], rows_v.at[b, pl.ds(0, sz)], gsem.at[b]

        def write_pair(row, half, b):
            sz = HALF_SZ[half]
            dst = out_hbm.at[xr0 + row, pl.ds(HALF_OFF[half], sz)]
            return rows_v.at[b, pl.ds(0, sz)], dst, wsem.at[b]

        def start_gather(row, half, b):
            src, dst, sem = gather_pair(row, half, b)
            pltpu.async_copy(src, dst, sem)

        def wait_gather(row, half, b):
            src, dst, sem = gather_pair(row, half, b)
            pltpu.make_async_copy(src, dst, sem).wait()

        def start_write(row, half, b):
            src, dst, sem = write_pair(row, half, b)
            pltpu.async_copy(src, dst, sem)

        def wait_write(row, half, b):
            src, dst, sem = write_pair(row, half, b)
            pltpu.make_async_copy(src, dst, sem).wait()

        start_gather(0, 0, 0)
        start_gather(0, 1, 1)
        start_gather(1, 0, 2)

        @pl.loop(0, rows_per_w // 2)
        def _grp(g):
            for kk in range(RING):
                row = 2 * g + (kk // 2)
                half = kk % 2
                wait_gather(row, half, kk)

                sz = HALF_SZ[half]
                buf = rows_v.at[kk]

                @pl.loop(0, sz, unroll=4)
                def _row(i):
                    for v in range(D_MODEL // L):
                        sl = pl.ds(v * L, L)
                        buf[i, sl] = buf[i, sl] * SCALE

                start_write(row, half, kk)

                kn = kk + 3
                row_n = 2 * g + (kn // 2)
                half_n = kn % 2
                bn = kn % RING
                jn = 4 * g + kn

                @pl.when(row_n < rows_per_w)
                def _():
                    @pl.when(jn >= RING)
                    def _():
                        wait_write(row_n - 2, half_n, bn)

                    start_gather(row_n, half_n, bn)

        for kk in range(RING):
            wait_write(rows_per_w - 2 + (kk // 2), kk % 2, kk)

    return k(x, table)


def kernel(x, table):
    return _sc_embed(x.astype(jnp.int32), table)
